# baseline (speedup 1.0000x reference)
import functools

import jax
import jax.numpy as jnp
from jax import lax
from jax.experimental import pallas as pl
from jax.experimental.pallas import tpu as pltpu
from jax.experimental.pallas import tpu_sc as plsc

NW = 32
CHUNK = 1000


def _rep_gather(vec, idx):
    return lax.gather(
        vec,
        idx[:, None],
        dimension_numbers=lax.GatherDimensionNumbers(
            offset_dims=(), collapsed_slice_dims=(0,), start_index_map=(0,)),
        slice_sizes=(1,),
        mode=lax.GatherScatterMode.PROMISE_IN_BOUNDS,
    )


def kernel(edge_index, node_type):
    E = edge_index.shape[1]
    T = node_type.shape[1]
    assert T == 4, "kernel specialized for 4 node-type features"
    assert E % NW == 0
    per_w = E // NW
    assert per_w % CHUNK == 0
    n_chunks = per_w // CHUNK
    assert n_chunks % 2 == 0 and n_chunks >= 4

    mesh = plsc.VectorSubcoreMesh(core_axis_name="c", subcore_axis_name="s")

    @functools.partial(
        pl.kernel,
        mesh=mesh,
        compiler_params=pltpu.CompilerParams(use_tc_tiling_on_sc=False),
        out_type=jax.ShapeDtypeStruct((E, T * T), jnp.float32),
        scratch_types=(
            [pltpu.VMEM((CHUNK,), jnp.int32)] * 4
            + [pltpu.VMEM((CHUNK, 16), jnp.float32)] * 4
            + [pltpu.VMEM((CHUNK, 16), jnp.float32)] * 2
            + [pltpu.SemaphoreType.DMA] * 6
        ),
    )
    def sc_kernel(edge0_hbm, edge1_hbm, table_hbm, out_hbm,
                  i0a, i0b, i1a, i1b, r0a, r0b, r1a, r1b, oa, ob,
                  sia, sib, sra, srb, soa, sob):
        idx0, idx1 = [i0a, i0b], [i1a, i1b]
        rows0, rows1 = [r0a, r0b], [r1a, r1b]
        outv = [oa, ob]
        s_idx, s_rows, s_out = [sia, sib], [sra, srb], [soa, sob]

        wid = lax.axis_index("s") * 2 + lax.axis_index("c")
        lane = lax.iota(jnp.int32, 16)
        hi = lax.shift_right_logical(lane, 2)
        base0 = wid * per_w

        def issue_idx(c, b):
            base = base0 + c * CHUNK
            pltpu.async_copy(edge0_hbm.at[pl.ds(base, CHUNK)], idx0[b],
                             s_idx[b])
            pltpu.async_copy(edge1_hbm.at[pl.ds(base, CHUNK)], idx1[b],
                             s_idx[b])

        def wait_idx(b):
            pltpu.make_async_copy(edge0_hbm.at[pl.ds(0, CHUNK)], idx0[b],
                                  s_idx[b]).wait()
            pltpu.make_async_copy(edge1_hbm.at[pl.ds(0, CHUNK)], idx1[b],
                                  s_idx[b]).wait()

        def issue_rows(b):
            pltpu.async_copy(table_hbm.at[idx0[b]], rows0[b], s_rows[b])
            pltpu.async_copy(table_hbm.at[idx1[b]], rows1[b], s_rows[b])

        def wait_rows(b):
            pltpu.make_async_copy(table_hbm.at[pl.ds(0, CHUNK)], rows0[b],
                                  s_rows[b]).wait()
            pltpu.make_async_copy(table_hbm.at[pl.ds(0, CHUNK)], rows1[b],
                                  s_rows[b]).wait()

        def issue_out(c, b):
            base = base0 + c * CHUNK
            pltpu.async_copy(outv[b], out_hbm.at[pl.ds(base, CHUNK)], s_out[b])

        def wait_out(b):
            pltpu.make_async_copy(outv[b], out_hbm.at[pl.ds(0, CHUNK)],
                                  s_out[b]).wait()

        def compute(b):
            r0, r1, ov = rows0[b], rows1[b], outv[b]

            @plsc.parallel_loop(0, CHUNK, 1, unroll=8)
            def edge_body(e):
                t0 = r0[e, :]
                t1 = r1[e, :]
                mx_t = jnp.maximum(t0, t1)
                mn_t = jnp.minimum(t0, t1)
                ov[e, :] = _rep_gather(mn_t, hi) * mx_t

        issue_idx(0, 0)
        wait_idx(0)
        issue_rows(0)
        issue_idx(1, 1)

        def pair_body(i, carry):
            for b in range(2):
                nb = 1 - b
                c = 2 * i + b
                @pl.when(c + 1 < n_chunks)
                def _():
                    wait_idx(nb)
                    issue_rows(nb)

                @pl.when(c >= 2)
                def _():
                    wait_out(b)

                wait_rows(b)
                compute(b)
                issue_out(c, b)

                @pl.when(c + 2 < n_chunks)
                def _():
                    issue_idx(c + 2, b)
            return carry

        lax.fori_loop(0, n_chunks // 2, pair_body, 0)
        wait_out(0)
        wait_out(1)

    table16 = jnp.tile(node_type, (1, 4))
    e0 = jnp.bitwise_and(edge_index[0], jnp.int32(0x7FFFFFFF))
    e1 = jnp.bitwise_and(edge_index[1], jnp.int32(0x7FFFFFFF))
    return sc_kernel(e0, e1, table16)


# speedup vs baseline: 1.9223x; 1.9223x over previous
import functools

import jax
import jax.numpy as jnp
from jax import lax
from jax.experimental import pallas as pl
from jax.experimental.pallas import tpu as pltpu
from jax.experimental.pallas import tpu_sc as plsc

NW = 32
CT = 8
CHUNK = CT * 128


def _rep_gather(vec, idx):
    return lax.gather(
        vec,
        idx[:, None],
        dimension_numbers=lax.GatherDimensionNumbers(
            offset_dims=(), collapsed_slice_dims=(0,), start_index_map=(0,)),
        slice_sizes=(1,),
        mode=lax.GatherScatterMode.PROMISE_IN_BOUNDS,
    )


def kernel(edge_index, node_type):
    E = edge_index.shape[1]
    T = node_type.shape[1]
    assert T == 4, "kernel specialized for 4 node-type features"
    assert E % CHUNK == 0
    NTILE = E // 128
    n_chunks = E // CHUNK
    PLANE = NTILE * 1024
    k_iters = -(-n_chunks // NW)
    n_pairs = -(-k_iters // 2)

    mesh = plsc.VectorSubcoreMesh(core_axis_name="c", subcore_axis_name="s")

    @functools.partial(
        pl.kernel,
        mesh=mesh,
        compiler_params=pltpu.CompilerParams(use_tc_tiling_on_sc=False,
                                             needs_layout_passes=False),
        out_type=jax.ShapeDtypeStruct((E * T * T,), jnp.float32),
        scratch_types=(
            [pltpu.VMEM((CHUNK,), jnp.int32)] * 4
            + [pltpu.VMEM((CHUNK, 16), jnp.float32)] * 4
            + [pltpu.VMEM((2 * CT * 1024,), jnp.float32)] * 2
            + [pltpu.SemaphoreType.DMA] * 6
        ),
    )
    def sc_kernel(edge0_hbm, edge1_hbm, table_hbm, out_hbm,
                  i0a, i0b, i1a, i1b, r0a, r0b, r1a, r1b, oa, ob,
                  sia, sib, sra, srb, soa, sob):
        idx0, idx1 = [i0a, i0b], [i1a, i1b]
        rows0, rows1 = [r0a, r0b], [r1a, r1b]
        outv = [oa, ob]
        s_idx, s_rows, s_out = [sia, sib], [sra, srb], [soa, sob]

        wid = lax.axis_index("s") * 2 + lax.axis_index("c")
        lane = lax.iota(jnp.int32, 16)
        hi = lax.shift_right_logical(lane, 2)
        pat = (lax.shift_right_logical(lane, 3) * (CT * 1024)
               + lax.bitwise_and(lane, 7) * 128)

        def chunk_of(k):
            return k * NW + wid

        def issue_idx(k, b):
            base = chunk_of(k) * CHUNK
            pltpu.async_copy(edge0_hbm.at[pl.ds(base, CHUNK)], idx0[b],
                             s_idx[b])
            pltpu.async_copy(edge1_hbm.at[pl.ds(base, CHUNK)], idx1[b],
                             s_idx[b])

        def wait_idx(b):
            pltpu.make_async_copy(edge0_hbm.at[pl.ds(0, CHUNK)], idx0[b],
                                  s_idx[b]).wait()
            pltpu.make_async_copy(edge1_hbm.at[pl.ds(0, CHUNK)], idx1[b],
                                  s_idx[b]).wait()

        def issue_rows(b):
            pltpu.async_copy(table_hbm.at[idx0[b]], rows0[b], s_rows[b])
            pltpu.async_copy(table_hbm.at[idx1[b]], rows1[b], s_rows[b])

        def wait_rows(b):
            pltpu.make_async_copy(table_hbm.at[pl.ds(0, CHUNK)], rows0[b],
                                  s_rows[b]).wait()
            pltpu.make_async_copy(table_hbm.at[pl.ds(0, CHUNK)], rows1[b],
                                  s_rows[b]).wait()

        def issue_out(k, b):
            j0 = chunk_of(k) * CT
            for p in range(2):
                pltpu.async_copy(
                    outv[b].at[pl.ds(p * CT * 1024, CT * 1024)],
                    out_hbm.at[pl.ds((p * NTILE + j0) * 1024, CT * 1024)],
                    s_out[b])

        def wait_out(b):
            for p in range(2):
                pltpu.make_async_copy(
                    outv[b].at[pl.ds(p * CT * 1024, CT * 1024)],
                    out_hbm.at[pl.ds(p * 1024, CT * 1024)],
                    s_out[b]).wait()

        def compute(b):
            r0, r1, ov = rows0[b], rows1[b], outv[b]

            @plsc.parallel_loop(0, CHUNK, 1, unroll=8)
            def edge_body(e):
                t0 = r0[e, :]
                t1 = r1[e, :]
                mx_t = jnp.maximum(t0, t1)
                mn_t = jnp.minimum(t0, t1)
                val = _rep_gather(mn_t, hi) * mx_t
                s = (lax.shift_right_logical(e, 7) * 1024
                     + lax.bitwise_and(e, 127))
                plsc.store_scatter(ov, [pat + s], val)

        def exists(k):
            return chunk_of(k) < n_chunks

        issue_idx(0, 0)
        wait_idx(0)
        issue_rows(0)
        issue_idx(1, 1)

        def pair_body(i, carry):
            for b in range(2):
                nb = 1 - b
                k = 2 * i + b

                @pl.when(exists(k + 1))
                def _():
                    wait_idx(nb)
                    issue_rows(nb)

                @pl.when((k >= 2) & exists(k))
                def _():
                    wait_out(b)

                @pl.when(exists(k))
                def _():
                    wait_rows(b)
                    compute(b)
                    issue_out(k, b)

                @pl.when(exists(k + 2))
                def _():
                    issue_idx(k + 2, b)
            return carry

        lax.fori_loop(0, n_pairs, pair_body, 0)
        wait_out(0)
        wait_out(1)

    table16 = jnp.tile(node_type, (1, 4))
    e0 = jnp.bitwise_and(edge_index[0], jnp.int32(0x7FFFFFFF))
    e1 = jnp.bitwise_and(edge_index[1], jnp.int32(0x7FFFFFFF))
    out1d = sc_kernel(e0, e1, table16)
    out4d = out1d.reshape(2, NTILE, 8, 128)
    return out4d.transpose((1, 3, 0, 2)).reshape(E, T * T)


# speedup vs baseline: 1.9301x; 1.0041x over previous
import functools

import jax
import jax.numpy as jnp
from jax import lax
from jax.experimental import pallas as pl
from jax.experimental.pallas import tpu as pltpu
from jax.experimental.pallas import tpu_sc as plsc

NW = 32
CT = 8
CHUNK = CT * 128


def _rep_gather(vec, idx):
    return lax.gather(
        vec,
        idx[:, None],
        dimension_numbers=lax.GatherDimensionNumbers(
            offset_dims=(), collapsed_slice_dims=(0,), start_index_map=(0,)),
        slice_sizes=(1,),
        mode=lax.GatherScatterMode.PROMISE_IN_BOUNDS,
    )


def kernel(edge_index, node_type):
    E = edge_index.shape[1]
    T = node_type.shape[1]
    assert T == 4, "kernel specialized for 4 node-type features"
    assert E % CHUNK == 0
    NTILE = E // 128
    n_chunks = E // CHUNK
    k_iters = -(-n_chunks // NW)
    n_pairs = -(-k_iters // 2)

    mesh = plsc.VectorSubcoreMesh(core_axis_name="c", subcore_axis_name="s")

    @functools.partial(
        pl.kernel,
        mesh=mesh,
        compiler_params=pltpu.CompilerParams(use_tc_tiling_on_sc=False,
                                             needs_layout_passes=False),
        out_type=jax.ShapeDtypeStruct((E * T * T,), jnp.float32),
        scratch_types=(
            [pltpu.VMEM((CHUNK,), jnp.int32)] * 4
            + [pltpu.VMEM((CHUNK, 16), jnp.float32)] * 4
            + [pltpu.VMEM((2 * CT * 1024,), jnp.float32)] * 2
            + [pltpu.SemaphoreType.DMA] * 6
        ),
    )
    def sc_kernel(edge0_hbm, edge1_hbm, table_hbm, out_hbm,
                  i0a, i0b, i1a, i1b, r0a, r0b, r1a, r1b, oa, ob,
                  sia, sib, sra, srb, soa, sob):
        idx0, idx1 = [i0a, i0b], [i1a, i1b]
        rows0, rows1 = [r0a, r0b], [r1a, r1b]
        outv = [oa, ob]
        s_idx, s_rows, s_out = [sia, sib], [sra, srb], [soa, sob]

        wid = lax.axis_index("s") * 2 + lax.axis_index("c")
        lane = lax.iota(jnp.int32, 16)
        hi = lax.shift_right_logical(lane, 2)
        pat = (lax.shift_right_logical(lane, 3) * (CT * 1024)
               + lax.bitwise_and(lane, 7) * 128)

        def chunk_of(k):
            return k * NW + wid

        def issue_idx(k, b):
            base = chunk_of(k) * CHUNK
            pltpu.async_copy(edge0_hbm.at[pl.ds(base, CHUNK)], idx0[b],
                             s_idx[b])
            pltpu.async_copy(edge1_hbm.at[pl.ds(base, CHUNK)], idx1[b],
                             s_idx[b])

        def wait_idx(b):
            pltpu.make_async_copy(edge0_hbm.at[pl.ds(0, CHUNK)], idx0[b],
                                  s_idx[b]).wait()
            pltpu.make_async_copy(edge1_hbm.at[pl.ds(0, CHUNK)], idx1[b],
                                  s_idx[b]).wait()

        def issue_rows(b):
            pltpu.async_copy(table_hbm.at[idx0[b]], rows0[b], s_rows[b])
            pltpu.async_copy(table_hbm.at[idx1[b]], rows1[b], s_rows[b])

        def wait_rows(b):
            pltpu.make_async_copy(table_hbm.at[pl.ds(0, CHUNK)], rows0[b],
                                  s_rows[b]).wait()
            pltpu.make_async_copy(table_hbm.at[pl.ds(0, CHUNK)], rows1[b],
                                  s_rows[b]).wait()

        def issue_out(k, b):
            j0 = chunk_of(k) * CT
            for p in range(2):
                pltpu.async_copy(
                    outv[b].at[pl.ds(p * CT * 1024, CT * 1024)],
                    out_hbm.at[pl.ds((p * NTILE + j0) * 1024, CT * 1024)],
                    s_out[b])

        def wait_out(b):
            for p in range(2):
                pltpu.make_async_copy(
                    outv[b].at[pl.ds(p * CT * 1024, CT * 1024)],
                    out_hbm.at[pl.ds(p * 1024, CT * 1024)],
                    s_out[b]).wait()

        def compute(b):
            r0, r1, ov = rows0[b], rows1[b], outv[b]

            @plsc.parallel_loop(0, CHUNK, 1, unroll=8)
            def edge_body(e):
                t0 = r0[e, :]
                t1 = r1[e, :]
                mx_t = jnp.maximum(t0, t1)
                mn_t = jnp.minimum(t0, t1)
                val = _rep_gather(mn_t, hi) * mx_t
                s = (lax.shift_right_logical(e, 7) * 1024
                     + lax.bitwise_and(e, 127))
                plsc.store_scatter(ov, [pat + s], val)

        def exists(k):
            return chunk_of(k) < n_chunks

        issue_idx(0, 0)
        wait_idx(0)
        issue_rows(0)
        issue_idx(1, 1)

        def pair_body(i, carry):
            for b in range(2):
                nb = 1 - b
                k = 2 * i + b

                @pl.when(exists(k + 1))
                def _():
                    wait_idx(nb)
                    issue_rows(nb)

                @pl.when((k >= 2) & exists(k))
                def _():
                    wait_out(b)

                @pl.when(exists(k))
                def _():
                    wait_rows(b)
                    compute(b)
                    issue_out(k, b)

                @pl.when(exists(k + 2))
                def _():
                    issue_idx(k + 2, b)
            return carry

        lax.fori_loop(0, n_pairs, pair_body, 0)
        wait_out(0)
        wait_out(1)

    table16 = jnp.tile(node_type, (1, 4))
    e0 = jnp.bitwise_and(edge_index[0], jnp.int32(0x7FFFFFFF))
    e1 = jnp.bitwise_and(edge_index[1], jnp.int32(0x7FFFFFFF))
    out1d = sc_kernel(e0, e1, table16)
    out4d = out1d.reshape(2, NTILE, 8, 128)
    return out4d.transpose((1, 3, 0, 2)).reshape(E, T * T)
